# baseline (speedup 1.0000x reference)
import functools

import jax
import jax.numpy as jnp
from jax import lax
from jax.experimental import pallas as pl
from jax.experimental.pallas import tpu as pltpu
from jax.experimental.pallas import tpu_sc as plsc


def _sc_broadcast_add(x_table, y_table, B, X, Y, D):
    info = plsc.get_sparse_core_info()
    NC, NS, L = info.num_cores, info.num_subcores, info.num_lanes
    mesh = plsc.VectorSubcoreMesh(core_axis_name="c", subcore_axis_name="s")

    @functools.partial(
        pl.kernel,
        mesh=mesh,
        out_type=jax.ShapeDtypeStruct((B, X, Y, D), jnp.float32),
        scratch_types=[
            pltpu.VMEM((Y, D), jnp.float32),
            pltpu.VMEM((1, D), jnp.float32),
            pltpu.VMEM((Y, D), jnp.float32),
            pltpu.SemaphoreType.DMA,
        ],
    )
    def k(x_hbm, y_hbm, out_hbm, yblk, xrow, blk, sem):
        wid = lax.axis_index("s") * NC + lax.axis_index("c")
        x = wid
        pltpu.sync_copy(y_hbm.at[pl.ds(0, Y)], yblk)
        pltpu.sync_copy(x_hbm.at[pl.ds(x, 1)], xrow)

        def row(y, carry):
            for d0 in range(D // L):
                sl = pl.ds(d0 * L, L)
                blk[y, sl] = yblk[y, sl] + xrow[0, sl]
            return carry

        lax.fori_loop(0, Y, row, 0)

        copies = [
            pltpu.async_copy(blk, out_hbm.at[b, x], sem) for b in range(B)
        ]
        for c in copies:
            c.wait()

    return k(x_table, y_table)


def kernel(inputs, x_table, y_table):
    B, X, Y, D = inputs.shape
    return _sc_broadcast_add(x_table, y_table, B, X, Y, D)


# speedup vs baseline: 1.4618x; 1.4618x over previous
import jax
import jax.numpy as jnp
from jax.experimental import pallas as pl
from jax.experimental.pallas import tpu as pltpu


def kernel(inputs, x_table, y_table):
    B, X, Y, D = inputs.shape

    def body(x_ref, y_ref, o_ref):
        xe = x_ref[:X]
        ye = y_ref[:Y]
        o_ref[...] = (xe[:, None, :] + ye[None, :, :])[None]

    return pl.pallas_call(
        body,
        grid=(B,),
        in_specs=[
            pl.BlockSpec(x_table.shape, lambda b: (0, 0)),
            pl.BlockSpec(y_table.shape, lambda b: (0, 0)),
        ],
        out_specs=pl.BlockSpec((1, X, Y, D), lambda b: (b, 0, 0, 0)),
        out_shape=jax.ShapeDtypeStruct((B, X, Y, D), jnp.float32),
    )(x_table, y_table)
